# baseline (speedup 1.0000x reference)
import jax
import jax.numpy as jnp
from jax.experimental import pallas as pl
from jax.experimental.pallas import tpu as pltpu

B, T, V, D, H = 2048, 32, 100, 128, 256


def _gru_kernel(idx_ref, h0_ref, len_ref, emb_ref, wihT_ref, whhT_ref,
                bih_ref, bhh_ref, out_ref):
    bias = bih_ref[...] + jnp.concatenate(
        [bhh_ref[:, :2 * H], jnp.zeros((1, H), jnp.float32)], axis=1)
    G = jnp.dot(emb_ref[...].astype(jnp.bfloat16), wihT_ref[...].astype(jnp.bfloat16),
                preferred_element_type=jnp.float32) + bias
    Gb = G.astype(jnp.bfloat16)
    whhT = whhT_ref[...].astype(jnp.bfloat16)
    bhh_n = bhh_ref[0, 2 * H:][None, :]
    lens = len_ref[...]
    idx = idx_ref[...]
    h = h0_ref[...]
    iota_v = jax.lax.broadcasted_iota(jnp.int32, (1, V), 1)

    for t in range(T):
        onehot = (idx[:, t][:, None] == iota_v).astype(jnp.bfloat16)
        gi = jnp.dot(onehot, Gb, preferred_element_type=jnp.float32)
        gh = jnp.dot(h.astype(jnp.bfloat16), whhT,
                     preferred_element_type=jnp.float32)
        r = jax.nn.sigmoid(gi[:, :H] + gh[:, :H])
        z = jax.nn.sigmoid(gi[:, H:2 * H] + gh[:, H:2 * H])
        n = jnp.tanh(gi[:, 2 * H:] + r * (gh[:, 2 * H:] + bhh_n))
        h = n + z * (h - n)
        out_ref[:, t, :] = jnp.where(t < lens, h, 0.0)


@jax.jit
def _run(output, h0, lens2d, emb, wihT, whhT, bih2d, bhh2d):
    BB = 128
    grid = (B // BB,)
    return pl.pallas_call(
        _gru_kernel,
        grid=grid,
        in_specs=[
            pl.BlockSpec((BB, T), lambda i: (i, 0)),
            pl.BlockSpec((BB, H), lambda i: (i, 0)),
            pl.BlockSpec((BB, 1), lambda i: (i, 0)),
            pl.BlockSpec((V, D), lambda i: (0, 0)),
            pl.BlockSpec((D, 3 * H), lambda i: (0, 0)),
            pl.BlockSpec((H, 3 * H), lambda i: (0, 0)),
            pl.BlockSpec((1, 3 * H), lambda i: (0, 0)),
            pl.BlockSpec((1, 3 * H), lambda i: (0, 0)),
        ],
        out_specs=pl.BlockSpec((BB, T, H), lambda i: (i, 0, 0)),
        out_shape=jax.ShapeDtypeStruct((B, T, H), jnp.float32),
        compiler_params=pltpu.CompilerParams(
            dimension_semantics=("parallel",)),
    )(output, h0, lens2d, emb, wihT, whhT, bih2d, bhh2d)


def kernel(output, conditioning, output_mask, output_word_len, emb,
           W_ih, W_hh, b_ih, b_hh):
    h0 = conditioning[0]
    lens2d = jnp.maximum(output_word_len, 1)[:, None].astype(jnp.int32)
    return _run(output.astype(jnp.int32), h0, lens2d, emb,
                W_ih.T, W_hh.T, b_ih[None, :], b_hh[None, :])


# speedup vs baseline: 1.3095x; 1.3095x over previous
import jax
import jax.numpy as jnp
from jax.experimental import pallas as pl
from jax.experimental.pallas import tpu as pltpu

B, T, V, D, H = 2048, 32, 100, 128, 256


def _gru_kernel(idx_ref, h0_ref, len_ref, emb_ref, wihT_ref, whhT_ref,
                bih_ref, bhh_ref, out_ref):
    bias = bih_ref[...] + jnp.concatenate(
        [bhh_ref[:, :2 * H], jnp.zeros((1, H), jnp.float32)], axis=1)
    G = jnp.dot(emb_ref[...].astype(jnp.bfloat16), wihT_ref[...].astype(jnp.bfloat16),
                preferred_element_type=jnp.float32) + bias
    Gb = G.astype(jnp.bfloat16)
    whhT = whhT_ref[...].astype(jnp.bfloat16)
    bhh_n = bhh_ref[0, 2 * H:][None, :]
    lens = len_ref[...]
    idx = idx_ref[...]
    h = h0_ref[...]
    iota_v = jax.lax.broadcasted_iota(jnp.int32, (1, V), 1)

    for t in range(T):
        onehot = (idx[:, t][:, None] == iota_v).astype(jnp.bfloat16)
        gi = jnp.dot(onehot, Gb, preferred_element_type=jnp.float32)
        gh = jnp.dot(h.astype(jnp.bfloat16), whhT,
                     preferred_element_type=jnp.float32)
        r = jax.nn.sigmoid(gi[:, :H] + gh[:, :H])
        z = jax.nn.sigmoid(gi[:, H:2 * H] + gh[:, H:2 * H])
        n = jnp.tanh(gi[:, 2 * H:] + r * (gh[:, 2 * H:] + bhh_n))
        h = n + z * (h - n)
        out_ref[:, t, :] = jnp.where(t < lens, h, 0.0)


@jax.jit
def _run(output, h0, lens2d, emb, wihT, whhT, bih2d, bhh2d):
    BB = 256
    grid = (B // BB,)
    return pl.pallas_call(
        _gru_kernel,
        grid=grid,
        in_specs=[
            pl.BlockSpec((BB, T), lambda i: (i, 0)),
            pl.BlockSpec((BB, H), lambda i: (i, 0)),
            pl.BlockSpec((BB, 1), lambda i: (i, 0)),
            pl.BlockSpec((V, D), lambda i: (0, 0)),
            pl.BlockSpec((D, 3 * H), lambda i: (0, 0)),
            pl.BlockSpec((H, 3 * H), lambda i: (0, 0)),
            pl.BlockSpec((1, 3 * H), lambda i: (0, 0)),
            pl.BlockSpec((1, 3 * H), lambda i: (0, 0)),
        ],
        out_specs=pl.BlockSpec((BB, T, H), lambda i: (i, 0, 0)),
        out_shape=jax.ShapeDtypeStruct((B, T, H), jnp.float32),
        compiler_params=pltpu.CompilerParams(
            dimension_semantics=("parallel",)),
    )(output, h0, lens2d, emb, wihT, whhT, bih2d, bhh2d)


def kernel(output, conditioning, output_mask, output_word_len, emb,
           W_ih, W_hh, b_ih, b_hh):
    h0 = conditioning[0]
    lens2d = jnp.maximum(output_word_len, 1)[:, None].astype(jnp.int32)
    return _run(output.astype(jnp.int32), h0, lens2d, emb,
                W_ih.T, W_hh.T, b_ih[None, :], b_hh[None, :])
